# baseline (speedup 1.0000x reference)
import functools

import jax
import jax.numpy as jnp
from jax import lax
from jax.experimental import pallas as pl
from jax.experimental.pallas import tpu as pltpu
from jax.experimental.pallas import tpu_sc as plsc

D = 64
NC, NS = 2, 16
NW = NC * NS
RG = 128
G = 5


def _make_lookup(n_rows: int):
    assert n_rows % (NW * RG * G) == 0
    r_total = n_rows // RG
    rpw = r_total // NW
    nch = rpw // G

    mesh = plsc.VectorSubcoreMesh(core_axis_name="c", subcore_axis_name="s")

    @functools.partial(
        pl.kernel,
        out_type=jax.ShapeDtypeStruct((r_total, RG, D), jnp.float32),
        mesh=mesh,
        scratch_types=[
            pltpu.VMEM((rpw, RG), jnp.int32),
            pltpu.VMEM((G, RG, D), jnp.float32),
            pltpu.SemaphoreType.DMA,
            pltpu.SemaphoreType.DMA,
        ],
        compiler_params=pltpu.CompilerParams(use_tc_tiling_on_sc=False),
    )
    def lookup(idx_hbm, table_hbm, out_hbm, idx_v, rows_v, idx_sem, gat_sem):
        wid = lax.axis_index("s") * NC + lax.axis_index("c")
        gbase = wid * rpw
        pltpu.async_copy(idx_hbm.at[pl.ds(gbase, rpw)], idx_v, idx_sem).wait()

        @pl.loop(0, nch)
        def _chunk(ch):
            waits = []
            for j in range(G):
                waits.append(pltpu.async_copy(
                    table_hbm.at[idx_v.at[ch * G + j]], rows_v.at[j], gat_sem))
            for w in waits:
                w.wait()
            pltpu.sync_copy(rows_v, out_hbm.at[pl.ds(gbase + ch * G, G)])

    return lookup


def kernel(x, emb_weight):
    b, t = x.shape
    n_rows = b * t
    idx = x.reshape(n_rows // RG, RG).astype(jnp.int32)
    out = _make_lookup(n_rows)(idx, emb_weight)
    return out.reshape(b, t, D)
